# baseline (speedup 1.0000x reference)
import functools

import jax
import jax.numpy as jnp
from jax import lax
from jax.experimental import pallas as pl
from jax.experimental.pallas import tpu as pltpu
from jax.experimental.pallas import tpu_sc as plsc

N_NODES = 100000
HIDDEN = 128
NUM_SEGMENTS = 1024

NUM_WORKERS = 32
NS = 16
SEG_PER_SUB = NUM_SEGMENTS // NS

CHUNK = 128
CPS = 8
NSLICES = 4
SLICE_CHUNKS = NUM_WORKERS * CPS
SLICE_ROWS = SLICE_CHUNKS * CHUNK
N_PAD = NSLICES * SLICE_ROWS
N_CHUNKS = N_PAD // CHUNK
LAST_FULL_CHUNK = N_NODES // CHUNK - 1
TAIL_CHUNK = 781
TAIL_ROWS = N_NODES - TAIL_CHUNK * CHUNK

TC_BLOCK = 1024
TC_BLOCKS_PER_SLICE = SLICE_ROWS // TC_BLOCK
LAST_REAL_BLOCK = (N_NODES - 1) // TC_BLOCK
NBUF = 4


def _gate_body(x_ref, w1_ref, b1_ref, w2t_ref, b2_ref, out_ref, *, g0):
    i = pl.program_id(0)
    x = x_ref[...]
    h = jnp.maximum(
        jnp.dot(x, w1_ref[...], preferred_element_type=jnp.float32) + b1_ref[...],
        0.0,
    )
    logit = jnp.sum(h * w2t_ref[...], axis=1, keepdims=True) + b2_ref[...]
    gate = jax.nn.sigmoid(logit).reshape(TC_BLOCK // HIDDEN, HIDDEN)
    row0 = (g0 + i) * TC_BLOCK
    rows = (row0
            + HIDDEN * lax.broadcasted_iota(jnp.int32, gate.shape, 0)
            + lax.broadcasted_iota(jnp.int32, gate.shape, 1))
    out_ref[...] = jnp.where(rows < N_NODES, gate, 0.0)


def _gate_slice(p, n_blocks, x, W1, b1t, w2t, b2m):
    g0 = p * TC_BLOCKS_PER_SLICE
    return pl.pallas_call(
        functools.partial(_gate_body, g0=g0),
        grid=(n_blocks,),
        in_specs=[
            pl.BlockSpec((TC_BLOCK, HIDDEN),
                         lambda i: (jnp.minimum(g0 + i, LAST_REAL_BLOCK), 0)),
            pl.BlockSpec((HIDDEN, HIDDEN), lambda i: (0, 0)),
            pl.BlockSpec((1, HIDDEN), lambda i: (0, 0)),
            pl.BlockSpec((1, HIDDEN), lambda i: (0, 0)),
            pl.BlockSpec((1, 1), lambda i: (0, 0)),
        ],
        out_specs=pl.BlockSpec((TC_BLOCK // HIDDEN, HIDDEN), lambda i: (i, 0)),
        out_shape=jax.ShapeDtypeStruct((SLICE_CHUNKS, HIDDEN), jnp.float32),
    )(x, W1, b1t, w2t, b2m)


def _mult_rows(buf, gate_v, j):
    base = (jnp.int32(j) if isinstance(j, int) else j) * CHUNK

    def mgroup(g, carry):
        gvec = gate_v[pl.ds(base + g * 16, 16)]
        r0 = g * 16
        for t in range(16):
            g16 = lax.broadcast(gvec[t], (16,))
            for k in range(HIDDEN // 16):
                sl = pl.ds(k * 16, 16)
                buf[r0 + t, sl] = buf[r0 + t, sl] * g16
        return carry

    lax.fori_loop(0, CHUNK // 16, mgroup, 0)


def _seed_and_finish(init_hbm, out_hbm, acc, c, s):
    pltpu.sync_copy(
        init_hbm.at[pl.ds(c * NUM_SEGMENTS + s * SEG_PER_SUB, SEG_PER_SUB)],
        acc.at[pl.ds(s * SEG_PER_SUB, SEG_PER_SUB)],
    )
    plsc.subcore_barrier()

    def finish():
        plsc.subcore_barrier()
        pltpu.sync_copy(
            acc.at[pl.ds(s * SEG_PER_SUB, SEG_PER_SUB)],
            out_hbm.at[pl.ds(c * NUM_SEGMENTS + s * SEG_PER_SUB, SEG_PER_SUB)],
        )
    return finish


def _make_seg_body(p):
    def body(x_hbm, gate_hbm, idx_hbm, init_hbm, out_hbm,
             idx_v, gate_v, bufs0, bufs1, bufs2, bufs3, acc,
             d0, d1, d2, d3, t0, t1, t2, t3):
        c = lax.axis_index("c")
        s = lax.axis_index("s")
        finish = _seed_and_finish(init_hbm, out_hbm, acc, c, s)
        w = c * NS + s
        bufs = (bufs0, bufs1, bufs2, bufs3)
        dsem = (d0, d1, d2, d3)
        tsem = (t0, t1, t2, t3)

        if p < NSLICES - 1:
            base_chunk = p * SLICE_CHUNKS + w * CPS
            pltpu.sync_copy(idx_hbm.at[pl.ds(base_chunk, CPS)], idx_v)
            pltpu.sync_copy(
                gate_hbm.at[pl.ds(w * CPS * CHUNK, CPS * CHUNK)], gate_v)

            def start_dma(j):
                return pltpu.async_copy(
                    x_hbm.at[pl.ds((base_chunk + j) * CHUNK, CHUNK)],
                    bufs[j % NBUF], dsem[j % NBUF])

            dma = [None] * CPS
            scat = [None] * CPS
            for j in range(NBUF):
                dma[j] = start_dma(j)
            for j in range(CPS):
                dma[j].wait()
                _mult_rows(bufs[j % NBUF], gate_v, j)
                scat[j] = pltpu.async_copy(
                    bufs[j % NBUF], acc.at[idx_v.at[j]], tsem[j % NBUF],
                    add=True)
                if 1 <= j and j + 3 < CPS:
                    scat[j - 1].wait()
                    dma[j + 3] = start_dma(j + 3)
            for j in range(CPS - NBUF, CPS):
                scat[j].wait()
        else:
            g_chunk = p * SLICE_CHUNKS + w
            idx_block = p * SLICE_CHUNKS + 8 * (w // 8)
            pltpu.sync_copy(idx_hbm.at[pl.ds(idx_block, 8)], idx_v)
            pltpu.sync_copy(
                gate_hbm.at[pl.ds(8 * (w // 8) * CHUNK, 8 * CHUNK)], gate_v)
            jj = w % 8
            is_full = g_chunk <= LAST_FULL_CHUNK
            is_tail = g_chunk == TAIL_CHUNK
            buf = bufs[0]

            @pl.when(is_full)
            def _():
                pltpu.sync_copy(x_hbm.at[pl.ds(g_chunk * CHUNK, CHUNK)], buf)

            @pl.when(is_tail)
            def _():
                zero16 = jnp.zeros((16,), jnp.float32)

                def zrow(r, carry):
                    for k in range(HIDDEN // 16):
                        buf[r, pl.ds(k * 16, 16)] = zero16
                    return carry

                lax.fori_loop(TAIL_ROWS, CHUNK, zrow, 0)
                pltpu.sync_copy(
                    x_hbm.at[pl.ds(g_chunk * CHUNK, TAIL_ROWS)],
                    buf.at[pl.ds(0, TAIL_ROWS)])

            @pl.when(is_full | is_tail)
            def _():
                _mult_rows(buf, gate_v, jj)
                pltpu.sync_copy(buf, acc.at[idx_v.at[jj]], add=True)

        finish()
    return body


def _seg_sum_slice(p, x, gate_p, idx_all, init):
    mesh = plsc.VectorSubcoreMesh(core_axis_name="c", subcore_axis_name="s")
    f = functools.partial(
        pl.kernel,
        mesh=mesh,
        out_type=jax.ShapeDtypeStruct((2 * NUM_SEGMENTS, HIDDEN), jnp.float32),
        scratch_types=[
            pltpu.VMEM((CPS, CHUNK), jnp.int32),
            pltpu.VMEM((CPS * CHUNK,), jnp.float32),
            pltpu.VMEM((CHUNK, HIDDEN), jnp.float32),
            pltpu.VMEM((CHUNK, HIDDEN), jnp.float32),
            pltpu.VMEM((CHUNK, HIDDEN), jnp.float32),
            pltpu.VMEM((CHUNK, HIDDEN), jnp.float32),
            pltpu.VMEM_SHARED((NUM_SEGMENTS, HIDDEN), jnp.float32),
            pltpu.SemaphoreType.DMA,
            pltpu.SemaphoreType.DMA,
            pltpu.SemaphoreType.DMA,
            pltpu.SemaphoreType.DMA,
            pltpu.SemaphoreType.DMA,
            pltpu.SemaphoreType.DMA,
            pltpu.SemaphoreType.DMA,
            pltpu.SemaphoreType.DMA,
        ],
    )(_make_seg_body(p))
    return f(x, gate_p.reshape(-1), idx_all, init)


def kernel(node_embeddings, batch_idx, W1, b1, W2, b2):
    idx = batch_idx.astype(jnp.int32)
    idx_pad = jnp.concatenate(
        [idx, jnp.zeros((N_PAD - N_NODES,), jnp.int32)]
    ).reshape(N_CHUNKS, CHUNK)

    b1t = b1.reshape(1, HIDDEN)
    w2t = W2.reshape(HIDDEN, 1).T
    b2m = b2.reshape(1, 1)

    gates = [_gate_slice(p, TC_BLOCKS_PER_SLICE if p < NSLICES - 1 else 2,
                         node_embeddings, W1, b1t, w2t, b2m)
             for p in range(NSLICES)]
    partial = jnp.zeros((2 * NUM_SEGMENTS, HIDDEN), jnp.float32)
    for p in range(NSLICES):
        partial = _seg_sum_slice(p, node_embeddings, gates[p], idx_pad, partial)
    return partial.reshape(2, NUM_SEGMENTS, HIDDEN).sum(axis=0)


# speedup vs baseline: 1.0021x; 1.0021x over previous
import functools

import jax
import jax.numpy as jnp
from jax import lax
from jax.experimental import pallas as pl
from jax.experimental.pallas import tpu as pltpu
from jax.experimental.pallas import tpu_sc as plsc

N_NODES = 100000
HIDDEN = 128
NUM_SEGMENTS = 1024

NUM_WORKERS = 32
NS = 16
SEG_PER_SUB = NUM_SEGMENTS // NS

CHUNK = 128
NSLICES = 5
CHUNKS_PER_SLICE = 5
IDX_STRIDE = 8
SLICE_ROWS = NUM_WORKERS * CHUNKS_PER_SLICE * CHUNK
WORKER_ROWS = CHUNKS_PER_SLICE * CHUNK
N_PAD = NSLICES * SLICE_ROWS

TC_BLOCK = 1024
TC_BLOCKS_PER_SLICE = SLICE_ROWS // TC_BLOCK
LAST_REAL_BLOCK = (N_NODES - 1) // TC_BLOCK


def _gate_body(x_ref, w1_ref, b1_ref, w2t_ref, b2_ref, out_ref, *, g0):
    i = pl.program_id(0)
    x = x_ref[...]
    h = jnp.maximum(
        jnp.dot(x, w1_ref[...], preferred_element_type=jnp.float32) + b1_ref[...],
        0.0,
    )
    logit = jnp.sum(h * w2t_ref[...], axis=1, keepdims=True) + b2_ref[...]
    gated = jax.nn.sigmoid(logit) * x
    row0 = (g0 + i) * TC_BLOCK
    rows = row0 + lax.broadcasted_iota(jnp.int32, (TC_BLOCK, 1), 0)
    out_ref[...] = jnp.where(rows < N_NODES, gated, 0.0)


def _gated_slice(p, x, W1, b1t, w2t, b2m):
    g0 = p * TC_BLOCKS_PER_SLICE
    return pl.pallas_call(
        functools.partial(_gate_body, g0=g0),
        grid=(TC_BLOCKS_PER_SLICE,),
        in_specs=[
            pl.BlockSpec((TC_BLOCK, HIDDEN),
                         lambda i: (jnp.minimum(g0 + i, LAST_REAL_BLOCK), 0)),
            pl.BlockSpec((HIDDEN, HIDDEN), lambda i: (0, 0)),
            pl.BlockSpec((1, HIDDEN), lambda i: (0, 0)),
            pl.BlockSpec((1, HIDDEN), lambda i: (0, 0)),
            pl.BlockSpec((1, 1), lambda i: (0, 0)),
        ],
        out_specs=pl.BlockSpec((TC_BLOCK, HIDDEN), lambda i: (i, 0)),
        out_shape=jax.ShapeDtypeStruct((SLICE_ROWS, HIDDEN), jnp.float32),
    )(x, W1, b1t, w2t, b2m)


def _make_seg_sum_body(p):
    def body(rows_hbm, idx_hbm, init_hbm, out_hbm, idx_v, row_a, row_b, acc,
             sem_a, sem_b):
        c = lax.axis_index("c")
        s = lax.axis_index("s")
        pltpu.sync_copy(
            init_hbm.at[pl.ds(c * NUM_SEGMENTS + s * SEG_PER_SUB, SEG_PER_SUB)],
            acc.at[pl.ds(s * SEG_PER_SUB, SEG_PER_SUB)],
        )
        plsc.subcore_barrier()
        w = c * NS + s
        row_base = w * WORKER_ROWS
        idx_row = (p * NUM_WORKERS + w) * IDX_STRIDE
        pltpu.sync_copy(idx_hbm.at[pl.ds(idx_row, IDX_STRIDE)], idx_v)
        bufs = (row_a, row_b)
        sems = (sem_a, sem_b)
        handles = [None] * CHUNKS_PER_SLICE
        handles[0] = pltpu.async_copy(
            rows_hbm.at[pl.ds(row_base, CHUNK)], bufs[0], sems[0])
        for j in range(CHUNKS_PER_SLICE):
            if j + 1 < CHUNKS_PER_SLICE:
                handles[j + 1] = pltpu.async_copy(
                    rows_hbm.at[pl.ds(row_base + (j + 1) * CHUNK, CHUNK)],
                    bufs[(j + 1) % 2], sems[(j + 1) % 2])
            handles[j].wait()
            pltpu.sync_copy(bufs[j % 2], acc.at[idx_v.at[j]], add=True)
        plsc.subcore_barrier()
        pltpu.sync_copy(
            acc.at[pl.ds(s * SEG_PER_SUB, SEG_PER_SUB)],
            out_hbm.at[pl.ds(c * NUM_SEGMENTS + s * SEG_PER_SUB, SEG_PER_SUB)],
        )
    return body


def _seg_sum_slice(p, rows, idx_all, init):
    mesh = plsc.VectorSubcoreMesh(core_axis_name="c", subcore_axis_name="s")
    f = functools.partial(
        pl.kernel,
        mesh=mesh,
        out_type=jax.ShapeDtypeStruct((2 * NUM_SEGMENTS, HIDDEN), jnp.float32),
        scratch_types=[
            pltpu.VMEM((IDX_STRIDE, CHUNK), jnp.int32),
            pltpu.VMEM((CHUNK, HIDDEN), jnp.float32),
            pltpu.VMEM((CHUNK, HIDDEN), jnp.float32),
            pltpu.VMEM_SHARED((NUM_SEGMENTS, HIDDEN), jnp.float32),
            pltpu.SemaphoreType.DMA,
            pltpu.SemaphoreType.DMA,
        ],
    )(_make_seg_sum_body(p))
    return f(rows, idx_all, init)


def kernel(node_embeddings, batch_idx, W1, b1, W2, b2):
    idx = batch_idx.astype(jnp.int32)
    idx_pad = jnp.concatenate(
        [idx, jnp.zeros((N_PAD - N_NODES,), jnp.int32)]
    ).reshape(NSLICES, NUM_WORKERS, CHUNKS_PER_SLICE, CHUNK)
    idx_pad = jnp.pad(
        idx_pad, ((0, 0), (0, 0), (0, IDX_STRIDE - CHUNKS_PER_SLICE), (0, 0))
    ).reshape(NSLICES * NUM_WORKERS * IDX_STRIDE, CHUNK)

    b1t = b1.reshape(1, HIDDEN)
    w2t = W2.reshape(HIDDEN, 1).T
    b2m = b2.reshape(1, 1)

    gated = [_gated_slice(p, node_embeddings, W1, b1t, w2t, b2m)
             for p in range(NSLICES)]
    partial = jnp.zeros((2 * NUM_SEGMENTS, HIDDEN), jnp.float32)
    for p in range(NSLICES):
        partial = _seg_sum_slice(p, gated[p], idx_pad, partial)
    return partial.reshape(2, NUM_SEGMENTS, HIDDEN).sum(axis=0)
